# baseline (speedup 1.0000x reference)
import functools

import jax
import jax.numpy as jnp
from jax import lax
from jax.experimental import pallas as pl
from jax.experimental.pallas import tpu as pltpu
from jax.experimental.pallas import tpu_sc as plsc

N = 16384
B = 64
NNZ_PER_ROW = 16
CH = 16
CHN = CH * NNZ_PER_ROW


def _make_kernel():
    info = plsc.get_sparse_core_info()
    nc, ns = info.num_cores, info.num_subcores
    nw = nc * ns
    rows_per_w = N // nw
    n_chunks = rows_per_w // CH
    halves_per_w = 2 * n_chunks

    mesh = plsc.VectorSubcoreMesh(core_axis_name="c", subcore_axis_name="s")

    @functools.partial(
        pl.kernel,
        out_type=jax.ShapeDtypeStruct((B, N), jnp.float32),
        mesh=mesh,
        compiler_params=pltpu.CompilerParams(use_tc_tiling_on_sc=False,
                                             needs_layout_passes=False),
        scratch_types=[
            pltpu.VMEM((halves_per_w, 128), jnp.int32),
            pltpu.VMEM((rows_per_w * NNZ_PER_ROW,), jnp.float32),
            pltpu.VMEM((rows_per_w,), jnp.float32),
            pltpu.VMEM((CHN, B), jnp.bfloat16),
            pltpu.VMEM((CHN, B), jnp.bfloat16),
            pltpu.VMEM((B, CH), jnp.float32),
            pltpu.VMEM((B, CH), jnp.float32),
            pltpu.SemaphoreType.DMA,
            pltpu.SemaphoreType.DMA,
            pltpu.SemaphoreType.DMA,
            pltpu.SemaphoreType.DMA,
        ],
    )
    def k(xt_hbm, col_hbm, val_hbm, bias_hbm, out_hbm,
          col_v, val_v, bias_v, gba, gbb, oba, obb, ga, gb, sa, sb):
        wid = lax.axis_index("s") * nc + lax.axis_index("c")
        row0 = wid * rows_per_w

        pltpu.sync_copy(col_hbm.at[pl.ds(wid * halves_per_w, halves_per_w)],
                        col_v)
        pltpu.sync_copy(val_hbm.at[pl.ds(row0 * NNZ_PER_ROW,
                                         rows_per_w * NNZ_PER_ROW)], val_v)
        pltpu.sync_copy(bias_hbm.at[pl.ds(row0, rows_per_w)], bias_v)

        def fire_gather(t, gbuf, sem):
            pltpu.async_copy(xt_hbm.at[col_v.at[2 * t]],
                             gbuf.at[pl.ds(0, 128)], sem)
            pltpu.async_copy(xt_hbm.at[col_v.at[2 * t + 1]],
                             gbuf.at[pl.ds(128, 128)], sem)

        def wait_gather(gbuf, sem):
            pltpu.make_async_copy(xt_hbm.at[pl.ds(0, 128)],
                                  gbuf.at[pl.ds(0, 128)], sem).wait()
            pltpu.make_async_copy(xt_hbm.at[pl.ds(0, 128)],
                                  gbuf.at[pl.ds(128, 128)], sem).wait()

        def wait_store(obuf, sem):
            pltpu.make_async_copy(obuf, out_hbm.at[:, pl.ds(0, CH)],
                                  sem).wait()

        two_iota = 2 * lax.iota(jnp.int32, 16)
        col_idx = [two_iota, two_iota + 1, two_iota + 32, two_iota + 33]

        def compute(t, gbuf, obuf, sem):
            bv = bias_v[pl.ds(t * CH, CH)]
            for i in range(CH):
                vv = val_v[pl.ds((t * CH + i) * NNZ_PER_ROW, 16)]
                bb = bv[i]
                accs = [jnp.full((16,), 0.0, jnp.float32) + bb
                        for _ in range(4)]
                for j in range(NNZ_PER_ROW):
                    w = vv[j]
                    g = i * NNZ_PER_ROW + j
                    lo = plsc.unpack(gbuf[g, pl.ds(0, 32)],
                                     format=plsc.PackFormat.INTERLEAVED)
                    hi = plsc.unpack(gbuf[g, pl.ds(32, 32)],
                                     format=plsc.PackFormat.INTERLEAVED)
                    for c, part in enumerate((lo[0], lo[1], hi[0], hi[1])):
                        accs[c] = accs[c] + w * part
                row_idx = jnp.full((16,), i, jnp.int32)
                for c in range(4):
                    plsc.store_scatter(obuf, [col_idx[c], row_idx], accs[c])
            pltpu.async_copy(obuf, out_hbm.at[:, pl.ds(row0 + t * CH, CH)],
                             sem)

        fire_gather(0, gba, ga)

        def body(tt, _):
            t0 = 2 * tt
            t1 = t0 + 1
            fire_gather(t1, gbb, gb)
            wait_gather(gba, ga)

            @pl.when(tt > 0)
            def _():
                wait_store(oba, sa)

            compute(t0, gba, oba, sa)

            @pl.when(tt < n_chunks // 2 - 1)
            def _():
                fire_gather(t0 + 2, gba, ga)

            wait_gather(gbb, gb)

            @pl.when(tt > 0)
            def _():
                wait_store(obb, sb)

            compute(t1, gbb, obb, sb)
            return ()

        lax.fori_loop(0, n_chunks // 2, body, ())
        wait_store(oba, sa)
        wait_store(obb, sb)

    return k


def kernel(X, values, bias, crow_indices, col_indices):
    del crow_indices
    xtb = X.T.astype(jnp.bfloat16).reshape(N, B)
    col2d = col_indices.reshape(-1, 128)
    return _make_kernel()(xtb, col2d, values, bias)


# speedup vs baseline: 1.1413x; 1.1413x over previous
import functools

import jax
import jax.numpy as jnp
from jax import lax
from jax.experimental import pallas as pl
from jax.experimental.pallas import tpu as pltpu
from jax.experimental.pallas import tpu_sc as plsc

N = 16384
B = 64
NNZ_PER_ROW = 16
CH = 16
CHN = CH * NNZ_PER_ROW
GRP = 4

_GATHER_DIM_NUMS = lax.GatherDimensionNumbers(
    offset_dims=(), collapsed_slice_dims=(0,), start_index_map=(0,))


def _splat_lane(vec, j):
    idx = jnp.full((16, 1), j, dtype=jnp.int32)
    return lax.gather(vec, idx, _GATHER_DIM_NUMS, slice_sizes=(1,),
                      mode=lax.GatherScatterMode.PROMISE_IN_BOUNDS)


def _make_kernel():
    info = plsc.get_sparse_core_info()
    nc, ns = info.num_cores, info.num_subcores
    nw = nc * ns
    rows_per_w = N // nw
    n_chunks = rows_per_w // CH
    halves_per_w = 2 * n_chunks

    mesh = plsc.VectorSubcoreMesh(core_axis_name="c", subcore_axis_name="s")

    @functools.partial(
        pl.kernel,
        out_type=jax.ShapeDtypeStruct((B, N), jnp.float32),
        mesh=mesh,
        compiler_params=pltpu.CompilerParams(use_tc_tiling_on_sc=False,
                                             needs_layout_passes=False),
        scratch_types=[
            pltpu.VMEM((halves_per_w, 128), jnp.int32),
            pltpu.VMEM((rows_per_w * NNZ_PER_ROW,), jnp.int32),
            pltpu.VMEM((rows_per_w,), jnp.float32),
            pltpu.VMEM((CHN, B), jnp.bfloat16),
            pltpu.VMEM((CHN, B), jnp.bfloat16),
            pltpu.VMEM((B, CH), jnp.float32),
            pltpu.VMEM((B, CH), jnp.float32),
            pltpu.SemaphoreType.DMA,
            pltpu.SemaphoreType.DMA,
            pltpu.SemaphoreType.DMA,
            pltpu.SemaphoreType.DMA,
        ],
    )
    def k(xt_hbm, col_hbm, val_hbm, bias_hbm, out_hbm,
          col_v, val_v, bias_v, gba, gbb, oba, obb, ga, gb, sa, sb):
        wid = lax.axis_index("s") * nc + lax.axis_index("c")
        row0 = wid * rows_per_w

        pltpu.sync_copy(col_hbm.at[pl.ds(wid * halves_per_w, halves_per_w)],
                        col_v)
        pltpu.sync_copy(val_hbm.at[pl.ds(row0 * NNZ_PER_ROW,
                                         rows_per_w * NNZ_PER_ROW)], val_v)
        pltpu.sync_copy(bias_hbm.at[pl.ds(row0, rows_per_w)], bias_v)

        def fire_gather(t, gbuf, sem):
            pltpu.async_copy(xt_hbm.at[col_v.at[2 * t]],
                             gbuf.at[pl.ds(0, 128)], sem)
            pltpu.async_copy(xt_hbm.at[col_v.at[2 * t + 1]],
                             gbuf.at[pl.ds(128, 128)], sem)

        def wait_gather(gbuf, sem):
            pltpu.make_async_copy(xt_hbm.at[pl.ds(0, 128)],
                                  gbuf.at[pl.ds(0, 128)], sem).wait()
            pltpu.make_async_copy(xt_hbm.at[pl.ds(0, 128)],
                                  gbuf.at[pl.ds(128, 128)], sem).wait()

        def wait_store(obuf, sem):
            pltpu.make_async_copy(obuf, out_hbm.at[:, pl.ds(0, CH)],
                                  sem).wait()

        two_iota = 2 * lax.iota(jnp.int32, 16)
        col_idx = [two_iota, two_iota + 1, two_iota + 32, two_iota + 33]

        def compute(t, gbuf, obuf, sem):
            bv = bias_v[pl.ds(t * CH, CH)]
            for i in range(CH):
                vvp = val_v[pl.ds((t * CH + i) * NNZ_PER_ROW, 16)]
                bb = _splat_lane(bv, i).astype(jnp.float32)
                accs = [bb, bb, bb, bb]
                for g0 in range(0, NNZ_PER_ROW, GRP):
                    s_lo = s_hi = None
                    for j in range(g0, g0 + GRP):
                        wv = plsc.bitcast(_splat_lane(vvp, j), jnp.bfloat16)
                        gr = i * NNZ_PER_ROW + j
                        p_lo = wv * gbuf[gr, pl.ds(0, 32)]
                        p_hi = wv * gbuf[gr, pl.ds(32, 32)]
                        s_lo = p_lo if s_lo is None else s_lo + p_lo
                        s_hi = p_hi if s_hi is None else s_hi + p_hi
                    lo = plsc.unpack(s_lo, format=plsc.PackFormat.INTERLEAVED)
                    hi = plsc.unpack(s_hi, format=plsc.PackFormat.INTERLEAVED)
                    for c, part in enumerate((lo[0], lo[1], hi[0], hi[1])):
                        accs[c] = accs[c] + part
                row_idx = jnp.full((16,), i, jnp.int32)
                for c in range(4):
                    plsc.store_scatter(obuf, [col_idx[c], row_idx], accs[c])
            pltpu.async_copy(obuf, out_hbm.at[:, pl.ds(row0 + t * CH, CH)],
                             sem)

        fire_gather(0, gba, ga)

        def body(tt, _):
            t0 = 2 * tt
            t1 = t0 + 1
            fire_gather(t1, gbb, gb)
            wait_gather(gba, ga)

            @pl.when(tt > 0)
            def _():
                wait_store(oba, sa)

            compute(t0, gba, oba, sa)

            @pl.when(tt < n_chunks // 2 - 1)
            def _():
                fire_gather(t0 + 2, gba, ga)

            wait_gather(gbb, gb)

            @pl.when(tt > 0)
            def _():
                wait_store(obb, sb)

            compute(t1, gbb, obb, sb)
            return ()

        lax.fori_loop(0, n_chunks // 2, body, ())
        wait_store(oba, sa)
        wait_store(obb, sb)

    return k


def kernel(X, values, bias, crow_indices, col_indices):
    del crow_indices
    xtb = X.T.astype(jnp.bfloat16).reshape(N, B)
    vb = values.astype(jnp.bfloat16)
    valp = lax.bitcast_convert_type(
        jnp.stack([vb, vb], axis=-1), jnp.int32)
    col2d = col_indices.reshape(-1, 128)
    return _make_kernel()(xtb, col2d, valp, bias)


# speedup vs baseline: 1.5173x; 1.3295x over previous
import functools

import jax
import jax.numpy as jnp
from jax import lax
from jax.experimental import pallas as pl
from jax.experimental.pallas import tpu as pltpu
from jax.experimental.pallas import tpu_sc as plsc

N = 16384
B = 64
NNZ_PER_ROW = 16
CH = 16
CHN = CH * NNZ_PER_ROW
GRP = 4

_GATHER_DIM_NUMS = lax.GatherDimensionNumbers(
    offset_dims=(), collapsed_slice_dims=(0,), start_index_map=(0,))


def _splat_lane(vec, j):
    idx = jnp.full((16, 1), j, dtype=jnp.int32)
    return lax.gather(vec, idx, _GATHER_DIM_NUMS, slice_sizes=(1,),
                      mode=lax.GatherScatterMode.PROMISE_IN_BOUNDS)


def _make_kernel():
    info = plsc.get_sparse_core_info()
    nc, ns = info.num_cores, info.num_subcores
    nw = nc * ns
    rows_per_w = N // nw
    n_chunks = rows_per_w // CH
    halves_per_w = 2 * n_chunks

    mesh = plsc.VectorSubcoreMesh(core_axis_name="c", subcore_axis_name="s")

    @functools.partial(
        pl.kernel,
        out_type=jax.ShapeDtypeStruct((B, N), jnp.float32),
        mesh=mesh,
        compiler_params=pltpu.CompilerParams(use_tc_tiling_on_sc=False,
                                             needs_layout_passes=False),
        scratch_types=[
            pltpu.VMEM((halves_per_w, 128), jnp.int32),
            pltpu.VMEM((rows_per_w * NNZ_PER_ROW,), jnp.float32),
            pltpu.VMEM((rows_per_w,), jnp.float32),
            pltpu.VMEM((CHN, B), jnp.bfloat16),
            pltpu.VMEM((CHN, B), jnp.bfloat16),
            pltpu.VMEM((B, CH), jnp.float32),
            pltpu.VMEM((B, CH), jnp.float32),
            pltpu.SemaphoreType.DMA,
            pltpu.SemaphoreType.DMA,
            pltpu.SemaphoreType.DMA,
            pltpu.SemaphoreType.DMA,
        ],
    )
    def k(xt_hbm, col_hbm, val_hbm, bias_hbm, out_hbm,
          col_v, val_v, bias_v, gba, gbb, oba, obb, ga, gb, sa, sb):
        wid = lax.axis_index("s") * nc + lax.axis_index("c")
        row0 = wid * rows_per_w

        pltpu.sync_copy(col_hbm.at[pl.ds(wid * halves_per_w, halves_per_w)],
                        col_v)
        pltpu.sync_copy(val_hbm.at[pl.ds(row0 * NNZ_PER_ROW,
                                         rows_per_w * NNZ_PER_ROW)], val_v)
        pltpu.sync_copy(bias_hbm.at[pl.ds(row0, rows_per_w)], bias_v)

        def fire_gather(t, gbuf, sem):
            pltpu.async_copy(xt_hbm.at[col_v.at[2 * t]],
                             gbuf.at[pl.ds(0, 128)], sem)
            pltpu.async_copy(xt_hbm.at[col_v.at[2 * t + 1]],
                             gbuf.at[pl.ds(128, 128)], sem)

        def wait_gather(gbuf, sem):
            pltpu.make_async_copy(xt_hbm.at[pl.ds(0, 128)],
                                  gbuf.at[pl.ds(0, 128)], sem).wait()
            pltpu.make_async_copy(xt_hbm.at[pl.ds(0, 128)],
                                  gbuf.at[pl.ds(128, 128)], sem).wait()

        def wait_store(obuf, sem):
            pltpu.make_async_copy(obuf, out_hbm.at[:, pl.ds(0, CH)],
                                  sem).wait()

        two_iota = 2 * lax.iota(jnp.int32, 16)
        col_idx = [two_iota, two_iota + 1, two_iota + 32, two_iota + 33]

        def compute(t, gbuf, obuf, sem):
            bv = bias_v[pl.ds(t * CH, CH)]

            @plsc.parallel_loop(0, CH, unroll=4)
            def _row(i):
                vv = val_v[pl.ds((t * CH + i) * NNZ_PER_ROW, 16)]
                bb = _splat_lane(bv, i)
                accs = [bb, bb, bb, bb]
                for g0 in range(0, NNZ_PER_ROW, GRP):
                    s_lo = s_hi = None
                    for j in range(g0, g0 + GRP):
                        wf = _splat_lane(vv, j)
                        wv = plsc.pack(wf, wf,
                                       format=plsc.PackFormat.INTERLEAVED)
                        gr = i * NNZ_PER_ROW + j
                        p_lo = wv * gbuf[gr, pl.ds(0, 32)]
                        p_hi = wv * gbuf[gr, pl.ds(32, 32)]
                        s_lo = p_lo if s_lo is None else s_lo + p_lo
                        s_hi = p_hi if s_hi is None else s_hi + p_hi
                    lo = plsc.unpack(s_lo, format=plsc.PackFormat.INTERLEAVED)
                    hi = plsc.unpack(s_hi, format=plsc.PackFormat.INTERLEAVED)
                    for c, part in enumerate((lo[0], lo[1], hi[0], hi[1])):
                        accs[c] = accs[c] + part
                row_idx = jnp.full((16,), i, jnp.int32)
                for c in range(4):
                    plsc.store_scatter(obuf, [col_idx[c], row_idx], accs[c])

            pltpu.async_copy(obuf, out_hbm.at[:, pl.ds(row0 + t * CH, CH)],
                             sem)

        fire_gather(0, gba, ga)

        def body(tt, _):
            t0 = 2 * tt
            t1 = t0 + 1
            fire_gather(t1, gbb, gb)
            wait_gather(gba, ga)

            @pl.when(tt > 0)
            def _():
                wait_store(oba, sa)

            compute(t0, gba, oba, sa)

            @pl.when(tt < n_chunks // 2 - 1)
            def _():
                fire_gather(t0 + 2, gba, ga)

            wait_gather(gbb, gb)

            @pl.when(tt > 0)
            def _():
                wait_store(obb, sb)

            compute(t1, gbb, obb, sb)
            return ()

        lax.fori_loop(0, n_chunks // 2, body, ())
        wait_store(oba, sa)
        wait_store(obb, sb)

    return k


def kernel(X, values, bias, crow_indices, col_indices):
    del crow_indices
    xtb = X.T.astype(jnp.bfloat16).reshape(N, B)
    col2d = col_indices.reshape(-1, 128)
    return _make_kernel()(xtb, col2d, values, bias)
